# Initial kernel scaffold; baseline (speedup 1.0000x reference)
import jax
import jax.numpy as jnp
from jax.experimental import pallas as pl


def kernel(edge_index, edge_values, emb_weight):
    raise NotImplementedError("write your pallas kernel here")


# speedup vs baseline: 1.4425x; 1.4425x over previous
import functools

import jax
import jax.numpy as jnp
from jax import lax
from jax.experimental import pallas as pl
from jax.experimental.pallas import tpu as pltpu
from jax.experimental.pallas import tpu_sc as plsc

N_USERS = 25000
N_NODES = 50000
D = 64
E = 800000
HALF = 25000

NC = 2
NS = 16
CHUNK = 1024
GSUB = 128
E_TILE = 50176
E_PAD = E_TILE * NS
N_CHUNKS = E_TILE // CHUNK
QUART = 12500
ACC_ROWS = 12800
DUMP = QUART


def _bcast_lane(vec, lane):
    idx = jnp.full((16, 1), lane, jnp.int32)
    dn = lax.GatherDimensionNumbers(
        offset_dims=(), collapsed_slice_dims=(0,), start_index_map=(0,))
    return lax.gather(vec, idx, dn, (1,),
                      mode=lax.GatherScatterMode.PROMISE_IN_BOUNDS)


def _spmm_body(col_hbm, row_hbm, val_hbm, x_hbm, out_hbm,
               rows_v, col_v, row_tmp, row_loc, val_v, acc, sem):
    c = lax.axis_index("c")
    s = lax.axis_index("s")
    tile_base = s * E_TILE

    def zbody(i, _):
        for k in range(4):
            rows_v[i, pl.ds(k * 16, 16)] = jnp.zeros((16,), jnp.float32)
        return 0
    lax.fori_loop(0, CHUNK, zbody, 0)

    for p in range(2):
        node_base = c * HALF + p * QUART

        zbase = pl.multiple_of(s * (ACC_ROWS // NS), 8)
        pltpu.sync_copy(rows_v.at[pl.ds(0, ACC_ROWS // NS)],
                        acc.at[pl.ds(zbase, ACC_ROWS // NS)])
        plsc.subcore_barrier()

        def chunk_body(t, _):
            base_e = pl.multiple_of(tile_base + t * CHUNK, 8)
            pltpu.sync_copy(col_hbm.at[pl.ds(base_e, CHUNK)], col_v)
            pltpu.sync_copy(row_hbm.at[pl.ds(base_e, CHUNK)], row_tmp)
            pltpu.sync_copy(val_hbm.at[pl.ds(base_e, CHUNK)], val_v)

            descs = [
                pltpu.async_copy(x_hbm.at[col_v.at[pl.ds(g * GSUB, GSUB)]],
                                 rows_v.at[pl.ds(g * GSUB, GSUB)], sem)
                for g in range(CHUNK // GSUB)
            ]
            for d_ in descs:
                d_.wait()

            def locbody(i, _):
                r = row_tmp[pl.ds(i * 16, 16)]
                loc = r - node_base
                inr = (loc >= 0) & (loc < QUART)
                loc = jnp.where(inr, loc, DUMP)
                row_loc[i // 8, pl.ds((i % 8) * 16, 16)] = loc
                return 0
            lax.fori_loop(0, CHUNK // 16, locbody, 0)

            def sbody(g, _):
                v16 = val_v[pl.ds(g * 16, 16)]
                for jj in range(16):
                    vb = _bcast_lane(v16, jj)
                    e = g * 16 + jj
                    for k in range(4):
                        rows_v[e, pl.ds(k * 16, 16)] = (
                            rows_v[e, pl.ds(k * 16, 16)] * vb)
                return 0
            lax.fori_loop(0, CHUNK // 16, sbody, 0)

            for g in range(CHUNK // GSUB):
                pltpu.sync_copy(rows_v.at[pl.ds(g * GSUB, GSUB)],
                                acc.at[row_loc.at[g]], add=True)
            return 0

        lax.fori_loop(0, N_CHUNKS, chunk_body, 0)

        plsc.subcore_barrier()

        @pl.when(s == 0)
        def _():
            pltpu.sync_copy(acc.at[pl.ds(0, QUART)],
                            out_hbm.at[pl.ds(node_base, QUART)])

        plsc.subcore_barrier()

        if p == 0:
            lax.fori_loop(0, CHUNK, zbody, 0)


_spmm = pl.kernel(
    _spmm_body,
    out_type=jax.ShapeDtypeStruct((N_NODES, D), jnp.float32),
    mesh=plsc.VectorSubcoreMesh(core_axis_name="c", subcore_axis_name="s"),
    scratch_types=[
        pltpu.VMEM((CHUNK, D), jnp.float32),
        pltpu.VMEM((CHUNK,), jnp.int32),
        pltpu.VMEM((CHUNK,), jnp.int32),
        pltpu.VMEM((CHUNK // GSUB, GSUB), jnp.int32),
        pltpu.VMEM((CHUNK,), jnp.float32),
        pltpu.VMEM_SHARED((ACC_ROWS, D), jnp.float32),
        pltpu.SemaphoreType.DMA,
    ],
    compiler_params=pltpu.CompilerParams(use_tc_tiling_on_sc=False),
)


def _combine_body(x0_ref, y1_ref, y2_ref, o_ref):
    o_ref[...] = (x0_ref[...] + 0.5 * y1_ref[...]
                  + (1.0 / 3.0) * y2_ref[...])


def _combine(x0, y1, y2):
    blk = 2000
    return pl.pallas_call(
        _combine_body,
        out_shape=jax.ShapeDtypeStruct((N_NODES, D), jnp.float32),
        grid=(N_NODES // blk,),
        in_specs=[pl.BlockSpec((blk, D), lambda i: (i, 0))] * 3,
        out_specs=pl.BlockSpec((blk, D), lambda i: (i, 0)),
    )(x0, y1, y2)


@jax.jit
def kernel(edge_index, edge_values, emb_weight):
    pad = E_PAD - E
    col = jnp.concatenate([edge_index[1], jnp.zeros((pad,), jnp.int32)])
    row = jnp.concatenate(
        [edge_index[0], jnp.full((pad,), N_NODES, jnp.int32)])
    val = jnp.concatenate([edge_values, jnp.zeros((pad,), jnp.float32)])

    y1 = _spmm(col, row, val, emb_weight)
    y2 = _spmm(col, row, val, y1)
    out = _combine(emb_weight, y1, y2)
    return (out[:N_USERS], out[N_USERS:])
